# baseline (speedup 1.0000x reference)
import functools

import jax
import jax.numpy as jnp
from jax import lax
from jax.experimental import pallas as pl
from jax.experimental.pallas import tpu as pltpu
from jax.experimental.pallas import tpu_sc as plsc

N_TRIAL = 2_000_000
N_INT = 128
N_CH = 16
H = 1.0 / N_INT
RMAX = 1.0
L = 16
NC, NS = 2, 16
NW = NC * NS
CH = 1600
NCHUNKS = N_TRIAL // CH
NK = (NCHUNKS + NW - 1) // NW
GROUPS = CH // L
NROW = N_INT + 1


def _tec_body(r_hbm, y_hbm, out_hbm, y_v, knots_v, mu_v, z_v, tab_v, r_buf,
              out_buf, sem_out, sem_r):
    c = lax.axis_index("c")
    s = lax.axis_index("s")
    wid = s * NC + c

    pltpu.sync_copy(y_hbm, y_v)

    iota = lax.iota(jnp.int32, L)
    zeros = jnp.zeros((L,), jnp.float32)

    iota_f = iota.astype(jnp.float32)
    for q in range(9):
        knots_v[pl.ds(q * L, L)] = (iota_f + float(q * L)) * H

    mu_v[0, :] = zeros
    z_v[0, :] = zeros

    def fwd(i, carry):
        muv, zv = carry
        xp = plsc.load_gather(knots_v, [jnp.full((L,), i + 1, jnp.int32)])
        xm = plsc.load_gather(knots_v, [jnp.full((L,), i - 1, jnp.int32)])
        yp = y_v[i + 1, :]
        yi = y_v[i, :]
        ym = y_v[i - 1, :]
        lv = 2.0 * (xp - xm) - H * muv
        mun = H / lv
        alpha = (3.0 / H) * (yp - yi) - (3.0 / H) * (yi - ym)
        zn = (alpha - H * zv) / lv
        mu_v[i, :] = mun
        z_v[i, :] = zn
        return (mun, zn)

    lax.fori_loop(1, N_INT, fwd, (zeros, zeros))

    def bwd(k, cnext):
        j = N_INT - 1 - k
        cj = z_v[j, :] - mu_v[j, :] * cnext
        yj = y_v[j, :]
        yj1 = y_v[j + 1, :]
        bj = (yj1 - yj) / H - H * (cnext + 2.0 * cj) / 3.0
        dj = (cnext - cj) / (3.0 * H)
        tab_v[j, :] = yj
        tab_v[NROW + j, :] = bj
        tab_v[2 * NROW + j, :] = cj
        tab_v[3 * NROW + j, :] = dj
        return cj

    lax.fori_loop(0, N_INT, bwd, zeros)
    for q in range(4):
        tab_v[q * NROW + N_INT, :] = zeros

    def out_copy(slot, chunk):
        return pltpu.make_async_copy(
            out_buf.at[pl.ds(slot * CH, CH)],
            out_hbm.at[pl.ds(chunk * CH, CH)],
            sem_out.at[slot])

    def r_copy(slot, chunk):
        return pltpu.make_async_copy(
            r_hbm.at[pl.ds(chunk * CH, CH)],
            r_buf.at[pl.ds(slot * CH, CH)],
            sem_r.at[slot])

    @pl.when(wid < NCHUNKS)
    def _():
        r_copy(0, wid).start()

    def do_chunk(it, chunk):
        slot = lax.rem(it, 2)

        @pl.when(it >= 2)
        def _():
            out_copy(slot, chunk - 2 * NW).wait()

        r_copy(slot, chunk).wait()
        nxt = chunk + NW

        @pl.when(nxt < NCHUNKS)
        def _():
            r_copy(lax.rem(it + 1, 2), nxt).start()

        rbase = slot * CH

        @plsc.parallel_loop(0, GROUPS, unroll=3)
        def grp(g):
            rv = r_buf[pl.ds(rbase + g * L, L)]
            idx = (rv * float(N_INT)).astype(jnp.int32)
            idxs = jnp.minimum(jnp.maximum(idx, 0), N_INT)
            drv = rv - idxs.astype(jnp.float32) * H
            row0 = slot * CH + g * L
            for p in range(L):
                b = idxs[p]
                d = drv[p]
                d2 = d * d
                av = tab_v[b, :]
                bv = tab_v[NROW + b, :]
                cv = tab_v[2 * NROW + b, :]
                dv = tab_v[3 * NROW + b, :]
                out_buf[row0 + p, :] = (av + d * bv) + d2 * (cv + d * dv)

        out_copy(slot, chunk).start()

    def chunk_iter(it, _):
        chunk = wid + it * NW

        @pl.when(chunk < NCHUNKS)
        def _():
            do_chunk(it, chunk)

        return 0

    lax.fori_loop(0, NK, chunk_iter, 0)

    for dit in (NK - 2, NK - 1):
        chunk = wid + dit * NW

        @pl.when(chunk < NCHUNKS)
        def _():
            out_copy(lax.rem(jnp.int32(dit), 2), chunk).wait()


def kernel(r_trial, r_knots, R_out, h, rmax):
    del r_knots, h, rmax
    mesh = plsc.VectorSubcoreMesh(
        core_axis_name="c", subcore_axis_name="s", num_cores=NC,
        num_subcores=NS)
    f = pl.kernel(
        _tec_body,
        out_type=jax.ShapeDtypeStruct((N_TRIAL, N_CH), jnp.float32),
        mesh=mesh,
        compiler_params=pltpu.CompilerParams(
            needs_layout_passes=False, use_tc_tiling_on_sc=False),
        scratch_types=[
            pltpu.VMEM((N_INT + 1, N_CH), jnp.float32),
            pltpu.VMEM((144,), jnp.float32),
            pltpu.VMEM((N_INT, N_CH), jnp.float32),
            pltpu.VMEM((N_INT, N_CH), jnp.float32),
            pltpu.VMEM((4 * NROW, N_CH), jnp.float32),
            pltpu.VMEM((2 * CH,), jnp.float32),
            pltpu.VMEM((2 * CH, N_CH), jnp.float32),
            pltpu.SemaphoreType.DMA((2,)),
            pltpu.SemaphoreType.DMA((2,)),
        ],
    )
    return f(r_trial, R_out)


# speedup vs baseline: 1.0207x; 1.0207x over previous
import functools

import jax
import jax.numpy as jnp
from jax import lax
from jax.experimental import pallas as pl
from jax.experimental.pallas import tpu as pltpu
from jax.experimental.pallas import tpu_sc as plsc

N_TRIAL = 2_000_000
N_INT = 128
N_CH = 16
H = 1.0 / N_INT
RMAX = 1.0
L = 16
NC, NS = 2, 16
NW = NC * NS
CH = 1600
NCHUNKS = N_TRIAL // CH
NK = (NCHUNKS + NW - 1) // NW
GROUPS = CH // L
NROW = N_INT + 1


def _tec_body(r_hbm, y_hbm, out_hbm, y_v, knots_v, mu_v, z_v, tab_v, r_buf,
              out_buf, sem_out, sem_r):
    c = lax.axis_index("c")
    s = lax.axis_index("s")
    wid = s * NC + c

    pltpu.sync_copy(y_hbm, y_v)

    iota = lax.iota(jnp.int32, L)
    zeros = jnp.zeros((L,), jnp.float32)

    iota_f = iota.astype(jnp.float32)
    for q in range(9):
        knots_v[pl.ds(q * L, L)] = (iota_f + float(q * L)) * H

    mu_v[0, :] = zeros
    z_v[0, :] = zeros

    def fwd(i, carry):
        muv, zv = carry
        xp = plsc.load_gather(knots_v, [jnp.full((L,), i + 1, jnp.int32)])
        xm = plsc.load_gather(knots_v, [jnp.full((L,), i - 1, jnp.int32)])
        yp = y_v[i + 1, :]
        yi = y_v[i, :]
        ym = y_v[i - 1, :]
        lv = 2.0 * (xp - xm) - H * muv
        mun = H / lv
        alpha = (3.0 / H) * (yp - yi) - (3.0 / H) * (yi - ym)
        zn = (alpha - H * zv) / lv
        mu_v[i, :] = mun
        z_v[i, :] = zn
        return (mun, zn)

    lax.fori_loop(1, N_INT, fwd, (zeros, zeros))

    def bwd(k, cnext):
        j = N_INT - 1 - k
        cj = z_v[j, :] - mu_v[j, :] * cnext
        yj = y_v[j, :]
        yj1 = y_v[j + 1, :]
        bj = (yj1 - yj) / H - H * (cnext + 2.0 * cj) / 3.0
        dj = (cnext - cj) / (3.0 * H)
        tab_v[j, :] = yj
        tab_v[NROW + j, :] = bj
        tab_v[2 * NROW + j, :] = cj
        tab_v[3 * NROW + j, :] = dj
        return cj

    lax.fori_loop(0, N_INT, bwd, zeros)
    for q in range(4):
        tab_v[q * NROW + N_INT, :] = zeros

    def out_copy(slot, chunk):
        return pltpu.make_async_copy(
            out_buf.at[pl.ds(slot * CH, CH)],
            out_hbm.at[pl.ds(chunk * CH, CH)],
            sem_out.at[slot])

    def r_copy(slot, chunk):
        return pltpu.make_async_copy(
            r_hbm.at[pl.ds(chunk * CH, CH)],
            r_buf.at[pl.ds(slot * CH, CH)],
            sem_r.at[slot])

    @pl.when(wid < NCHUNKS)
    def _():
        r_copy(0, wid).start()

    def do_chunk(it, chunk):
        slot = lax.rem(it, 2)

        @pl.when(it >= 2)
        def _():
            out_copy(slot, chunk - 2 * NW).wait()

        r_copy(slot, chunk).wait()
        nxt = chunk + NW

        @pl.when(nxt < NCHUNKS)
        def _():
            r_copy(lax.rem(it + 1, 2), nxt).start()

        rbase = slot * CH

        @plsc.parallel_loop(0, GROUPS, unroll=2)
        def grp(g):
            rv = r_buf[pl.ds(rbase + g * L, L)]
            idx = (rv * float(N_INT)).astype(jnp.int32)
            idxs = jnp.minimum(jnp.maximum(idx, 0), N_INT)
            drv = rv - idxs.astype(jnp.float32) * H
            row0 = slot * CH + g * L
            for p in range(L):
                b = idxs[p]
                d = drv[p]
                d2 = d * d
                av = tab_v[b, :]
                bv = tab_v[NROW + b, :]
                cv = tab_v[2 * NROW + b, :]
                dv = tab_v[3 * NROW + b, :]
                out_buf[row0 + p, :] = (av + d * bv) + d2 * (cv + d * dv)

        out_copy(slot, chunk).start()

    def chunk_iter(it, _):
        chunk = wid + it * NW

        @pl.when(chunk < NCHUNKS)
        def _():
            do_chunk(it, chunk)

        return 0

    lax.fori_loop(0, NK, chunk_iter, 0)

    for dit in (NK - 2, NK - 1):
        chunk = wid + dit * NW

        @pl.when(chunk < NCHUNKS)
        def _():
            out_copy(lax.rem(jnp.int32(dit), 2), chunk).wait()


def kernel(r_trial, r_knots, R_out, h, rmax):
    del r_knots, h, rmax
    mesh = plsc.VectorSubcoreMesh(
        core_axis_name="c", subcore_axis_name="s", num_cores=NC,
        num_subcores=NS)
    f = pl.kernel(
        _tec_body,
        out_type=jax.ShapeDtypeStruct((N_TRIAL, N_CH), jnp.float32),
        mesh=mesh,
        compiler_params=pltpu.CompilerParams(
            needs_layout_passes=False, use_tc_tiling_on_sc=False),
        scratch_types=[
            pltpu.VMEM((N_INT + 1, N_CH), jnp.float32),
            pltpu.VMEM((144,), jnp.float32),
            pltpu.VMEM((N_INT, N_CH), jnp.float32),
            pltpu.VMEM((N_INT, N_CH), jnp.float32),
            pltpu.VMEM((4 * NROW, N_CH), jnp.float32),
            pltpu.VMEM((2 * CH,), jnp.float32),
            pltpu.VMEM((2 * CH, N_CH), jnp.float32),
            pltpu.SemaphoreType.DMA((2,)),
            pltpu.SemaphoreType.DMA((2,)),
        ],
    )
    return f(r_trial, R_out)
